# baseline (speedup 1.0000x reference)
import functools

import jax
import jax.numpy as jnp
from jax import lax
from jax.experimental import pallas as pl
from jax.experimental.pallas import tpu as pltpu
from jax.experimental.pallas import tpu_sc as plsc

_N = 10000
_E = 320000
_D = 128
_G = 64
_NCLS = 10

_NWORK = 32
_EPW = _E // _NWORK
_GCH = 80
_NCHG = _EPW // _GCH
_NPW = 320
_NPAD = _NWORK * _NPW
_SCH = 4000
_NSCH = _E // _SCH
_RS = 64
_RSH = 6
_RINGR = 128
_FLR = 1024 // _RS
_LCAPR = 5024
_DPAD = _NPAD - 1
_NEG = -3.0e38
_NSLOT = 4


def _iota16():
    return lax.broadcasted_iota(jnp.int32, (16,), 0)


def _sc_mesh():
    return plsc.VectorSubcoreMesh(core_axis_name="c", subcore_axis_name="s")


def _edge_gather(P, Q, src, dst):
    @functools.partial(
        pl.kernel,
        out_type=jax.ShapeDtypeStruct((_E, _D), jnp.float32),
        mesh=_sc_mesh(),
        scratch_types=[
            pltpu.VMEM((_GCH,), jnp.int32),
            pltpu.VMEM((_GCH,), jnp.int32),
            pltpu.VMEM((_GCH, _D), jnp.float32),
            pltpu.VMEM((_GCH, _D), jnp.float32),
            pltpu.VMEM((_GCH,), jnp.int32),
            pltpu.VMEM((_GCH,), jnp.int32),
            pltpu.VMEM((_GCH, _D), jnp.float32),
            pltpu.VMEM((_GCH, _D), jnp.float32),
            pltpu.SemaphoreType.DMA,
            pltpu.SemaphoreType.DMA,
            pltpu.SemaphoreType.DMA,
            pltpu.SemaphoreType.DMA,
            pltpu.SemaphoreType.DMA,
            pltpu.SemaphoreType.DMA,
            pltpu.SemaphoreType.DMA,
            pltpu.SemaphoreType.DMA,
            pltpu.SemaphoreType.DMA,
            pltpu.SemaphoreType.DMA,
        ],
        compiler_params=pltpu.CompilerParams(needs_layout_passes=False),
    )
    def k(p_hbm, q_hbm, src_hbm, dst_hbm, r_hbm,
          sidx0, didx0, bq0, bp0, sidx1, didx1, bq1, bp1,
          isa0, isb0, ga0, gb0, ws0, isa1, isb1, ga1, gb1, ws1):
        sidx = [sidx0, sidx1]
        didx = [didx0, didx1]
        bq = [bq0, bq1]
        bp = [bp0, bp1]
        isa = [isa0, isa1]
        isb = [isb0, isb1]
        ga = [ga0, ga1]
        gb = [gb0, gb1]
        ws = [ws0, ws1]

        wid = lax.axis_index("s") * 2 + lax.axis_index("c")
        ebase = wid * _EPW

        def fire_idx(i, sl):
            @pl.when(i < _NCHG)
            def _():
                base = ebase + i * _GCH
                pltpu.async_copy(src_hbm.at[pl.ds(base, _GCH)], sidx[sl],
                                 isa[sl])
                pltpu.async_copy(dst_hbm.at[pl.ds(base, _GCH)], didx[sl],
                                 isb[sl])

        def fire_gather(i, sl):
            @pl.when(i < _NCHG)
            def _():
                pltpu.make_async_copy(src_hbm.at[pl.ds(ebase, _GCH)],
                                      sidx[sl], isa[sl]).wait()
                pltpu.make_async_copy(dst_hbm.at[pl.ds(ebase, _GCH)],
                                      didx[sl], isb[sl]).wait()
                pltpu.async_copy(q_hbm.at[sidx[sl]], bq[sl], ga[sl])
                pltpu.async_copy(p_hbm.at[didx[sl]], bp[sl], gb[sl])

        fire_idx(jnp.int32(0), 0)
        fire_idx(jnp.int32(1), 1)
        fire_gather(jnp.int32(0), 0)

        nblk = (_NCHG + 1) // 2

        def blk(bi, _):
            for b in range(2):
                i = bi * 2 + b
                sl = b
                so = 1 - b

                @pl.when(i < _NCHG)
                def _(i=i, sl=sl, so=so):
                    pltpu.make_async_copy(q_hbm.at[sidx[sl]], bq[sl],
                                          ga[sl]).wait()
                    pltpu.make_async_copy(p_hbm.at[didx[sl]], bp[sl],
                                          gb[sl]).wait()

                    def row(r, _):
                        for c in range(_D // 16):
                            pv = bp[sl][r, pl.ds(c * 16, 16)]
                            plsc.addupdate(bq[sl].at[r, pl.ds(c * 16, 16)],
                                           pv)
                        return 0

                    lax.fori_loop(0, _GCH, row, 0)
                    base = ebase + i * _GCH
                    pltpu.async_copy(bq[sl], r_hbm.at[pl.ds(base, _GCH)],
                                     ws[sl])
                    fire_idx(i + 2, sl)

                    @pl.when(i >= 1)
                    def _():
                        pltpu.make_async_copy(
                            bq[so], r_hbm.at[pl.ds(ebase, _GCH)],
                            ws[so]).wait()

                    fire_gather(i + 1, so)
            return 0

        lax.fori_loop(0, nblk, blk, 0)
        pltpu.make_async_copy(bq[(_NCHG - 1) % 2],
                              r_hbm.at[pl.ds(ebase, _GCH)],
                              ws[(_NCHG - 1) % 2]).wait()

    return k(P, Q, src, dst)


def _build_lists(dst):
    @functools.partial(
        pl.kernel,
        out_type=(
            jax.ShapeDtypeStruct((_NWORK, _LCAPR, _RS), jnp.int32),
            jax.ShapeDtypeStruct((_NWORK, _LCAPR, _RS), jnp.int32),
            jax.ShapeDtypeStruct((_NWORK * 16,), jnp.int32),
        ),
        mesh=_sc_mesh(),
        scratch_types=[
            pltpu.VMEM((_SCH,), jnp.int32),
            pltpu.VMEM((_RINGR, _RS), jnp.int32),
            pltpu.VMEM((_RINGR, _RS), jnp.int32),
        ],
        compiler_params=pltpu.CompilerParams(needs_layout_passes=False),
    )
    def k(dst_hbm, le_hbm, ld_hbm, cnt_hbm, dchunk, ring_e, ring_d):
        wid = lax.axis_index("s") * 2 + lax.axis_index("c")
        lo = wid * _NPW
        iot = _iota16()
        lo16 = jnp.full((16,), lo, jnp.int32)
        hi16 = lo16 + _NPW
        one16 = jnp.full((16,), 1, jnp.int32)
        s16 = jnp.full((16,), 16, jnp.int32)
        lane15 = jnp.full((16,), 15, jnp.int32)
        zero16 = jnp.zeros((16,), jnp.int32)
        dpad16 = jnp.full((16,), _DPAD, jnp.int32)

        def pre(i, _):
            for c in range(_RS // 16):
                ring_e[i, pl.ds(c * 16, 16)] = zero16
            for c in range(_RS // 16):
                ring_d[i, pl.ds(c * 16, 16)] = dpad16
            return 0

        lax.fori_loop(0, _RINGR, pre, 0)

        def chunk(ch, carry):
            off16, flushed = carry
            base = ch * _SCH
            with jax.named_scope("lists_scan_dma"):
                pltpu.sync_copy(dst_hbm.at[pl.ds(base, _SCH)], dchunk)
            eid0 = jnp.full((16,), base, jnp.int32) + iot

            def comp(i, cc):
                o16, eidv = cc
                d = dchunk[pl.ds(i * 16, 16)]
                m = (d >= lo16) & (d < hi16)
                cs = plsc.cumsum(m.astype(jnp.int32))
                pos = o16 + cs - one16
                rrow = lax.shift_right_logical(pos, _RSH) & (_RINGR - 1)
                rcol = pos & (_RS - 1)
                plsc.store_scatter(ring_e, [rrow, rcol], eidv, mask=m)
                plsc.store_scatter(ring_d, [rrow, rcol], d, mask=m)
                tail = cs.at[lane15].get(mode="promise_in_bounds")
                return o16 + tail, eidv + s16

            with jax.named_scope("lists_comp"):
                off16, _ = lax.fori_loop(0, _SCH // 16, comp, (off16, eid0))
            staged = jnp.max(off16)

            def flush(_, fl):
                cond = (staged - fl) >= 1024

                @pl.when(cond)
                def _():
                    rr = pl.multiple_of(
                        lax.shift_right_logical(fl, _RSH) & (_RINGR - 1),
                        _FLR)
                    fr = pl.multiple_of(
                        lax.shift_right_logical(fl, _RSH), _FLR)
                    pltpu.sync_copy(ring_e.at[pl.ds(rr, _FLR)],
                                    le_hbm.at[wid, pl.ds(fr, _FLR)])
                    pltpu.sync_copy(ring_d.at[pl.ds(rr, _FLR)],
                                    ld_hbm.at[wid, pl.ds(fr, _FLR)])

                return fl + lax.select(cond, jnp.int32(1024), jnp.int32(0))

            with jax.named_scope("lists_flush"):
                flushed = lax.fori_loop(0, 5, flush, flushed)
            return off16, flushed

        off16, flushed = lax.fori_loop(0, _NSCH, chunk,
                                       (zero16, jnp.int32(0)))
        rr = pl.multiple_of(
            lax.shift_right_logical(flushed, _RSH) & (_RINGR - 1), _FLR)
        fr = pl.multiple_of(lax.shift_right_logical(flushed, _RSH), _FLR)
        pltpu.sync_copy(ring_e.at[pl.ds(rr, _FLR)],
                        le_hbm.at[wid, pl.ds(fr, _FLR)])
        pltpu.sync_copy(ring_d.at[pl.ds(rr, _FLR)],
                        ld_hbm.at[wid, pl.ds(fr, _FLR)])
        dchunk[pl.ds(0, 16)] = off16
        pltpu.sync_copy(dchunk.at[pl.ds(0, 16)],
                        cnt_hbm.at[pl.ds(wid * 16, 16)])

    return k(dst)


def _segmax(M, le, ld, cnt):
    @functools.partial(
        pl.kernel,
        out_type=jax.ShapeDtypeStruct((_NPAD, _D), jnp.float32),
        mesh=_sc_mesh(),
        scratch_types=[
            pltpu.VMEM((_NPW, _D), jnp.float32),
            pltpu.VMEM((_NPW, _D), jnp.float32),
            pltpu.VMEM((16,), jnp.int32),
            pltpu.VMEM((_RS,), jnp.int32),
            pltpu.VMEM((_RS,), jnp.int32),
            pltpu.VMEM((_RS,), jnp.int32),
            pltpu.VMEM((_RS,), jnp.int32),
            pltpu.VMEM((_RS,), jnp.int32),
            pltpu.VMEM((_RS,), jnp.int32),
            pltpu.VMEM((_RS,), jnp.int32),
            pltpu.VMEM((_RS,), jnp.int32),
            pltpu.VMEM((_RS, _D), jnp.float32),
            pltpu.VMEM((_RS, _D), jnp.float32),
            pltpu.VMEM((_RS, _D), jnp.float32),
            pltpu.VMEM((_RS, _D), jnp.float32),
            pltpu.SemaphoreType.DMA,
            pltpu.SemaphoreType.DMA,
            pltpu.SemaphoreType.DMA,
            pltpu.SemaphoreType.DMA,
            pltpu.SemaphoreType.DMA,
            pltpu.SemaphoreType.DMA,
            pltpu.SemaphoreType.DMA,
            pltpu.SemaphoreType.DMA,
            pltpu.SemaphoreType.DMA,
            pltpu.SemaphoreType.DMA,
            pltpu.SemaphoreType.DMA,
            pltpu.SemaphoreType.DMA,
        ],
        compiler_params=pltpu.CompilerParams(needs_layout_passes=False),
    )
    def k(m_hbm, le_hbm, ld_hbm, cnt_hbm, h_hbm, agg, agg2, cbuf,
          lbe0, lbe1, lbe2, lbe3, lbd0, lbd1, lbd2, lbd3,
          mb0, mb1, mb2, mb3,
          ise0, ise1, ise2, ise3, isd0, isd1, isd2, isd3,
          gs0, gs1, gs2, gs3):
        lbe = [lbe0, lbe1, lbe2, lbe3]
        lbd = [lbd0, lbd1, lbd2, lbd3]
        mb = [mb0, mb1, mb2, mb3]
        ise = [ise0, ise1, ise2, ise3]
        isd = [isd0, isd1, isd2, isd3]
        gs = [gs0, gs1, gs2, gs3]

        wid = lax.axis_index("s") * 2 + lax.axis_index("c")
        lo = wid * _NPW
        iot = _iota16()
        lo16 = jnp.full((16,), lo, jnp.int32)
        hi16 = lo16 + _NPW
        one16 = jnp.full((16,), 1, jnp.int32)
        two16 = jnp.full((16,), 2, jnp.int32)
        neg = jnp.full((16,), _NEG, jnp.float32)
        zf = jnp.zeros((16,), jnp.float32)
        zero16 = jnp.zeros((16,), jnp.int32)
        col16 = [jnp.full((16,), c * 16, jnp.int32) + iot
                 for c in range(_D // 16)]

        def initrow(r, _):
            for c in range(_D // 16):
                agg[r, pl.ds(c * 16, 16)] = neg
            for c in range(_D // 16):
                agg2[r, pl.ds(c * 16, 16)] = neg
            return 0

        lax.fori_loop(0, _NPW, initrow, 0)

        pltpu.sync_copy(cnt_hbm.at[pl.ds(wid * 16, 16)], cbuf)
        kt = cbuf[pl.ds(0, 16)][0]
        nsub = lax.shift_right_logical(kt + (_RS - 1), _RSH)

        def fire_idx(j, b):
            @pl.when(j < nsub)
            def _():
                pltpu.async_copy(le_hbm.at[wid, j], lbe[b], ise[b])
                pltpu.async_copy(ld_hbm.at[wid, j], lbd[b], isd[b])

        def fire_gather(j, b):
            @pl.when(j < nsub)
            def _():
                pltpu.make_async_copy(le_hbm.at[wid, j], lbe[b],
                                      ise[b]).wait()
                pltpu.async_copy(m_hbm.at[lbe[b]], mb[b], gs[b])

        for b in range(_NSLOT):
            fire_idx(jnp.int32(b), b)
        for b in range(_NSLOT - 1):
            fire_gather(jnp.int32(b), b)

        nblk = lax.shift_right_logical(nsub + (_NSLOT - 1), 2)

        def blk(bi, _):
            for b in range(_NSLOT):
                j = bi * _NSLOT + b
                fire_gather(j + (_NSLOT - 1), (b + _NSLOT - 1) % _NSLOT)

                @pl.when(j < nsub)
                def _(b=b, j=j):
                    pltpu.make_async_copy(ld_hbm.at[wid, j], lbd[b],
                                          isd[b]).wait()
                    with jax.named_scope("segmax_gwait"):
                        pltpu.make_async_copy(m_hbm.at[lbe[b]], mb[b],
                                              gs[b]).wait()

                    def acc(i, carry):
                        pa, pb = carry
                        dva = plsc.load_gather(lbd[b], [pa])
                        dvb = plsc.load_gather(lbd[b], [pb])
                        oka = (dva >= lo16) & (dva < hi16)
                        okb = (dvb >= lo16) & (dvb < hi16)
                        lia = dva - lo16
                        lib = dvb - lo16
                        for c in range(_D // 16):
                            aia = [lia, col16[c]]
                            aib = [lib, col16[c]]
                            cura = plsc.load_gather(agg, aia, mask=oka)
                            curb = plsc.load_gather(agg2, aib, mask=okb)
                            mva = mb[b][2 * i, pl.ds(c * 16, 16)]
                            mvb = mb[b][2 * i + 1, pl.ds(c * 16, 16)]
                            plsc.store_scatter(agg, aia,
                                               jnp.maximum(cura, mva),
                                               mask=oka)
                            plsc.store_scatter(agg2, aib,
                                               jnp.maximum(curb, mvb),
                                               mask=okb)
                        return pa + two16, pb + two16

                    with jax.named_scope("segmax_acc"):
                        lax.fori_loop(0, _RS // 2, acc, (zero16, one16))

                fire_idx(j + _NSLOT, b)
            return 0

        lax.fori_loop(0, nblk, blk, 0)

        def outrow(r, _):
            for c in range(_D // 16):
                agg[r, pl.ds(c * 16, 16)] = jnp.maximum(
                    jnp.maximum(agg[r, pl.ds(c * 16, 16)],
                                agg2[r, pl.ds(c * 16, 16)]), zf)
            return 0

        lax.fori_loop(0, _NPW, outrow, 0)
        pltpu.sync_copy(agg, h_hbm.at[pl.ds(lo, _NPW)])

    return k(M, le, ld, cnt)


def _fc0_bn_pq(x, W0, b0, g, bt, Wa, Wb, be1):
    def body(x_ref, w0_ref, b0_ref, g_ref, bt_ref, wa_ref, wb_ref, be1_ref,
             p_ref, q_ref):
        h = jnp.dot(x_ref[...], w0_ref[...],
                    preferred_element_type=jnp.float32) + b0_ref[...]
        mu = jnp.mean(h, axis=0, keepdims=True)
        var = jnp.mean((h - mu) ** 2, axis=0, keepdims=True)
        hn = (h - mu) * lax.rsqrt(var + 1e-5) * g_ref[...] + bt_ref[...]
        p_ref[...] = jnp.dot(hn, wa_ref[...],
                             preferred_element_type=jnp.float32) + be1_ref[...]
        q_ref[...] = jnp.dot(hn, wb_ref[...],
                             preferred_element_type=jnp.float32)

    return pl.pallas_call(
        body,
        out_shape=(jax.ShapeDtypeStruct((_N, _D), jnp.float32),
                   jax.ShapeDtypeStruct((_N, _D), jnp.float32)),
    )(x, W0, b0, g, bt, Wa, Wb, be1)


def _bn_pq(h, g, bt, Wa, Wb, be1):
    def body(h_ref, g_ref, bt_ref, wa_ref, wb_ref, be1_ref, p_ref, q_ref):
        h = h_ref[...]
        mu = jnp.mean(h, axis=0, keepdims=True)
        var = jnp.mean((h - mu) ** 2, axis=0, keepdims=True)
        hn = (h - mu) * lax.rsqrt(var + 1e-5) * g_ref[...] + bt_ref[...]
        p_ref[...] = jnp.dot(hn, wa_ref[...],
                             preferred_element_type=jnp.float32) + be1_ref[...]
        q_ref[...] = jnp.dot(hn, wb_ref[...],
                             preferred_element_type=jnp.float32)

    return pl.pallas_call(
        body,
        out_shape=(jax.ShapeDtypeStruct((_N, _D), jnp.float32),
                   jax.ShapeDtypeStruct((_N, _D), jnp.float32)),
    )(h, g, bt, Wa, Wb, be1)


def _edge_mlp(R, We2, be2):
    br = 512
    grid = _E // br

    def body(r_ref, w_ref, b_ref, o_ref):
        h = jnp.maximum(r_ref[...], 0.0).astype(jnp.bfloat16)
        w = w_ref[...].astype(jnp.bfloat16)
        o_ref[...] = jnp.dot(h, w,
                             preferred_element_type=jnp.float32) + b_ref[...]

    return pl.pallas_call(
        body,
        grid=(grid,),
        in_specs=[pl.BlockSpec((br, _D), lambda i: (i, 0)),
                  pl.BlockSpec((_D, _D), lambda i: (0, 0)),
                  pl.BlockSpec((1, _D), lambda i: (0, 0))],
        out_specs=pl.BlockSpec((br, _D), lambda i: (i, 0)),
        out_shape=jax.ShapeDtypeStruct((_E, _D), jnp.float32),
    )(R, We2, be2)


def _head(h1, h2, batch2d, W1, b1, W2p, b2p):
    def body(h1_ref, h2_ref, b_ref, w1_ref, b1_ref, w2_ref, b2_ref, o_ref):
        bt = b_ref[...]
        gidx = lax.broadcasted_iota(jnp.int32, (_G, _N), 0)
        oh = (bt == gidx).astype(jnp.float32)
        s1 = jnp.dot(oh, h1_ref[...], preferred_element_type=jnp.float32)
        s2 = jnp.dot(oh, h2_ref[...], preferred_element_type=jnp.float32)
        cnt = jnp.maximum(jnp.sum(oh, axis=1, keepdims=True), 1.0)
        pooled = jnp.concatenate([s1, s2], axis=1) / cnt
        z = jnp.maximum(
            jnp.dot(pooled, w1_ref[...],
                    preferred_element_type=jnp.float32) + b1_ref[...], 0.0)
        lg = jnp.dot(z, w2_ref[...],
                     preferred_element_type=jnp.float32) + b2_ref[...]
        mx = jnp.max(lg, axis=1, keepdims=True)
        ls = jnp.log(jnp.sum(jnp.exp(lg - mx), axis=1, keepdims=True))
        o_ref[...] = lg - mx - ls

    return pl.pallas_call(
        body,
        out_shape=jax.ShapeDtypeStruct((_G, 128), jnp.float32),
    )(h1, h2, batch2d, W1, b1, W2p, b2p)


def kernel(x, edge_index, batch, W0, b0, bn_g0, bn_b0, We1_0, be1_0, We2_0,
           be2_0, bn_g1, bn_b1, We1_1, be1_1, We2_1, be2_1, W1, b1, W2, b2):
    f32 = jnp.float32
    src = edge_index[0]
    dst = edge_index[1]
    r2 = lambda v: v.reshape(1, -1)

    le, ld, cnt = _build_lists(dst)

    Wa0 = We1_0[:_D] - We1_0[_D:]
    Wb0 = We1_0[_D:]
    P0, Q0 = _fc0_bn_pq(x, W0, r2(b0), r2(bn_g0), r2(bn_b0), Wa0, Wb0,
                        r2(be1_0))
    R0 = _edge_gather(P0, Q0, src, dst)
    M0 = _edge_mlp(R0, We2_0, r2(be2_0))
    h1 = _segmax(M0, le, ld, cnt)[:_N]

    Wa1 = We1_1[:_D] - We1_1[_D:]
    Wb1 = We1_1[_D:]
    P1, Q1 = _bn_pq(h1, r2(bn_g1), r2(bn_b1), Wa1, Wb1, r2(be1_1))
    R1 = _edge_gather(P1, Q1, src, dst)
    M1 = _edge_mlp(R1, We2_1, r2(be2_1))
    h2 = _segmax(M1, le, ld, cnt)[:_N]

    W2p = jnp.zeros((256, 128), f32).at[:, :_NCLS].set(W2)
    b2p = jnp.full((1, 128), -1e30, f32).at[0, :_NCLS].set(b2)
    out = _head(h1, h2, batch.reshape(1, _N).astype(jnp.int32), W1, r2(b1),
                W2p, b2p)
    return out[:, :_NCLS]


# speedup vs baseline: 1.0052x; 1.0052x over previous
import functools

import jax
import jax.numpy as jnp
from jax import lax
from jax.experimental import pallas as pl
from jax.experimental.pallas import tpu as pltpu
from jax.experimental.pallas import tpu_sc as plsc

_N = 10000
_E = 320000
_D = 128
_G = 64
_NCLS = 10

_NWORK = 32
_EPW = _E // _NWORK
_GCH = 80
_NCHG = _EPW // _GCH
_NPW = 320
_NPAD = _NWORK * _NPW
_SCH = 4000
_NSCH = _E // _SCH
_RS = 64
_RSH = 6
_RINGR = 128
_FLR = 1024 // _RS
_LCAPR = 5024
_DPAD = _NPAD - 1
_NEG = -3.0e38
_NSLOT = 4


def _iota16():
    return lax.broadcasted_iota(jnp.int32, (16,), 0)


def _sc_mesh():
    return plsc.VectorSubcoreMesh(core_axis_name="c", subcore_axis_name="s")


def _edge_gather(P, Q, src, dst):
    @functools.partial(
        pl.kernel,
        out_type=jax.ShapeDtypeStruct((_E, _D), jnp.float32),
        mesh=_sc_mesh(),
        scratch_types=[
            pltpu.VMEM((_GCH,), jnp.int32),
            pltpu.VMEM((_GCH,), jnp.int32),
            pltpu.VMEM((_GCH, _D), jnp.float32),
            pltpu.VMEM((_GCH, _D), jnp.float32),
            pltpu.VMEM((_GCH,), jnp.int32),
            pltpu.VMEM((_GCH,), jnp.int32),
            pltpu.VMEM((_GCH, _D), jnp.float32),
            pltpu.VMEM((_GCH, _D), jnp.float32),
            pltpu.SemaphoreType.DMA,
            pltpu.SemaphoreType.DMA,
            pltpu.SemaphoreType.DMA,
            pltpu.SemaphoreType.DMA,
            pltpu.SemaphoreType.DMA,
            pltpu.SemaphoreType.DMA,
            pltpu.SemaphoreType.DMA,
            pltpu.SemaphoreType.DMA,
            pltpu.SemaphoreType.DMA,
            pltpu.SemaphoreType.DMA,
        ],
        compiler_params=pltpu.CompilerParams(needs_layout_passes=False),
    )
    def k(p_hbm, q_hbm, src_hbm, dst_hbm, r_hbm,
          sidx0, didx0, bq0, bp0, sidx1, didx1, bq1, bp1,
          isa0, isb0, ga0, gb0, ws0, isa1, isb1, ga1, gb1, ws1):
        sidx = [sidx0, sidx1]
        didx = [didx0, didx1]
        bq = [bq0, bq1]
        bp = [bp0, bp1]
        isa = [isa0, isa1]
        isb = [isb0, isb1]
        ga = [ga0, ga1]
        gb = [gb0, gb1]
        ws = [ws0, ws1]

        wid = lax.axis_index("s") * 2 + lax.axis_index("c")
        ebase = wid * _EPW

        def fire_idx(i, sl):
            @pl.when(i < _NCHG)
            def _():
                base = ebase + i * _GCH
                pltpu.async_copy(src_hbm.at[pl.ds(base, _GCH)], sidx[sl],
                                 isa[sl])
                pltpu.async_copy(dst_hbm.at[pl.ds(base, _GCH)], didx[sl],
                                 isb[sl])

        def fire_gather(i, sl):
            @pl.when(i < _NCHG)
            def _():
                pltpu.make_async_copy(src_hbm.at[pl.ds(ebase, _GCH)],
                                      sidx[sl], isa[sl]).wait()
                pltpu.make_async_copy(dst_hbm.at[pl.ds(ebase, _GCH)],
                                      didx[sl], isb[sl]).wait()
                pltpu.async_copy(q_hbm.at[sidx[sl]], bq[sl], ga[sl])
                pltpu.async_copy(p_hbm.at[didx[sl]], bp[sl], gb[sl])

        fire_idx(jnp.int32(0), 0)
        fire_idx(jnp.int32(1), 1)
        fire_gather(jnp.int32(0), 0)

        nblk = (_NCHG + 1) // 2

        def blk(bi, _):
            for b in range(2):
                i = bi * 2 + b
                sl = b
                so = 1 - b

                @pl.when(i < _NCHG)
                def _(i=i, sl=sl, so=so):
                    pltpu.make_async_copy(q_hbm.at[sidx[sl]], bq[sl],
                                          ga[sl]).wait()
                    pltpu.make_async_copy(p_hbm.at[didx[sl]], bp[sl],
                                          gb[sl]).wait()

                    def row(r, _):
                        for c in range(_D // 16):
                            pv = bp[sl][r, pl.ds(c * 16, 16)]
                            plsc.addupdate(bq[sl].at[r, pl.ds(c * 16, 16)],
                                           pv)
                        return 0

                    lax.fori_loop(0, _GCH, row, 0)
                    base = ebase + i * _GCH
                    pltpu.async_copy(bq[sl], r_hbm.at[pl.ds(base, _GCH)],
                                     ws[sl])
                    fire_idx(i + 2, sl)

                    @pl.when(i >= 1)
                    def _():
                        pltpu.make_async_copy(
                            bq[so], r_hbm.at[pl.ds(ebase, _GCH)],
                            ws[so]).wait()

                    fire_gather(i + 1, so)
            return 0

        lax.fori_loop(0, nblk, blk, 0)
        pltpu.make_async_copy(bq[(_NCHG - 1) % 2],
                              r_hbm.at[pl.ds(ebase, _GCH)],
                              ws[(_NCHG - 1) % 2]).wait()

    return k(P, Q, src, dst)


def _build_lists(dst):
    @functools.partial(
        pl.kernel,
        out_type=(
            jax.ShapeDtypeStruct((_NWORK, _LCAPR, _RS), jnp.int32),
            jax.ShapeDtypeStruct((_NWORK, _LCAPR, _RS), jnp.int32),
            jax.ShapeDtypeStruct((_NWORK * 16,), jnp.int32),
        ),
        mesh=_sc_mesh(),
        scratch_types=[
            pltpu.VMEM((_SCH,), jnp.int32),
            pltpu.VMEM((_RINGR, _RS), jnp.int32),
            pltpu.VMEM((_RINGR, _RS), jnp.int32),
        ],
        compiler_params=pltpu.CompilerParams(needs_layout_passes=False),
    )
    def k(dst_hbm, le_hbm, ld_hbm, cnt_hbm, dchunk, ring_e, ring_d):
        wid = lax.axis_index("s") * 2 + lax.axis_index("c")
        lo = wid * _NPW
        iot = _iota16()
        lo16 = jnp.full((16,), lo, jnp.int32)
        hi16 = lo16 + _NPW
        one16 = jnp.full((16,), 1, jnp.int32)
        s16 = jnp.full((16,), 16, jnp.int32)
        lane15 = jnp.full((16,), 15, jnp.int32)
        zero16 = jnp.zeros((16,), jnp.int32)
        dpad16 = jnp.full((16,), _DPAD, jnp.int32)

        def pre(i, _):
            for c in range(_RS // 16):
                ring_e[i, pl.ds(c * 16, 16)] = zero16
            for c in range(_RS // 16):
                ring_d[i, pl.ds(c * 16, 16)] = dpad16
            return 0

        lax.fori_loop(0, _RINGR, pre, 0)

        def chunk(ch, carry):
            off16, flushed = carry
            base = ch * _SCH
            with jax.named_scope("lists_scan_dma"):
                pltpu.sync_copy(dst_hbm.at[pl.ds(base, _SCH)], dchunk)
            eid0 = jnp.full((16,), base, jnp.int32) + iot

            def comp(i, cc):
                o16, eidv = cc
                d = dchunk[pl.ds(i * 16, 16)]
                m = (d >= lo16) & (d < hi16)
                cs = plsc.cumsum(m.astype(jnp.int32))
                pos = o16 + cs - one16
                rrow = lax.shift_right_logical(pos, _RSH) & (_RINGR - 1)
                rcol = pos & (_RS - 1)
                plsc.store_scatter(ring_e, [rrow, rcol], eidv, mask=m)
                plsc.store_scatter(ring_d, [rrow, rcol], d, mask=m)
                tail = cs.at[lane15].get(mode="promise_in_bounds")
                return o16 + tail, eidv + s16

            with jax.named_scope("lists_comp"):
                off16, _ = lax.fori_loop(0, _SCH // 16, comp, (off16, eid0))
            staged = jnp.max(off16)

            def flush(_, fl):
                cond = (staged - fl) >= 1024

                @pl.when(cond)
                def _():
                    rr = pl.multiple_of(
                        lax.shift_right_logical(fl, _RSH) & (_RINGR - 1),
                        _FLR)
                    fr = pl.multiple_of(
                        lax.shift_right_logical(fl, _RSH), _FLR)
                    pltpu.sync_copy(ring_e.at[pl.ds(rr, _FLR)],
                                    le_hbm.at[wid, pl.ds(fr, _FLR)])
                    pltpu.sync_copy(ring_d.at[pl.ds(rr, _FLR)],
                                    ld_hbm.at[wid, pl.ds(fr, _FLR)])

                return fl + lax.select(cond, jnp.int32(1024), jnp.int32(0))

            with jax.named_scope("lists_flush"):
                flushed = lax.fori_loop(0, 5, flush, flushed)
            return off16, flushed

        off16, flushed = lax.fori_loop(0, _NSCH, chunk,
                                       (zero16, jnp.int32(0)))
        rr = pl.multiple_of(
            lax.shift_right_logical(flushed, _RSH) & (_RINGR - 1), _FLR)
        fr = pl.multiple_of(lax.shift_right_logical(flushed, _RSH), _FLR)
        pltpu.sync_copy(ring_e.at[pl.ds(rr, _FLR)],
                        le_hbm.at[wid, pl.ds(fr, _FLR)])
        pltpu.sync_copy(ring_d.at[pl.ds(rr, _FLR)],
                        ld_hbm.at[wid, pl.ds(fr, _FLR)])
        dchunk[pl.ds(0, 16)] = off16
        pltpu.sync_copy(dchunk.at[pl.ds(0, 16)],
                        cnt_hbm.at[pl.ds(wid * 16, 16)])

    return k(dst)


def _segmax(M, le, ld, cnt):
    @functools.partial(
        pl.kernel,
        out_type=jax.ShapeDtypeStruct((_NPAD, _D), jnp.float32),
        mesh=_sc_mesh(),
        scratch_types=[
            pltpu.VMEM((_NPW, _D), jnp.float32),
            pltpu.VMEM((_NPW, _D), jnp.float32),
            pltpu.VMEM((16,), jnp.int32),
            pltpu.VMEM((_RS,), jnp.int32),
            pltpu.VMEM((_RS,), jnp.int32),
            pltpu.VMEM((_RS,), jnp.int32),
            pltpu.VMEM((_RS,), jnp.int32),
            pltpu.VMEM((_RS,), jnp.int32),
            pltpu.VMEM((_RS,), jnp.int32),
            pltpu.VMEM((_RS,), jnp.int32),
            pltpu.VMEM((_RS,), jnp.int32),
            pltpu.VMEM((_RS, _D), jnp.float32),
            pltpu.VMEM((_RS, _D), jnp.float32),
            pltpu.VMEM((_RS, _D), jnp.float32),
            pltpu.VMEM((_RS, _D), jnp.float32),
            pltpu.SemaphoreType.DMA,
            pltpu.SemaphoreType.DMA,
            pltpu.SemaphoreType.DMA,
            pltpu.SemaphoreType.DMA,
            pltpu.SemaphoreType.DMA,
            pltpu.SemaphoreType.DMA,
            pltpu.SemaphoreType.DMA,
            pltpu.SemaphoreType.DMA,
            pltpu.SemaphoreType.DMA,
            pltpu.SemaphoreType.DMA,
            pltpu.SemaphoreType.DMA,
            pltpu.SemaphoreType.DMA,
        ],
        compiler_params=pltpu.CompilerParams(needs_layout_passes=False),
    )
    def k(m_hbm, le_hbm, ld_hbm, cnt_hbm, h_hbm, agg, agg2, cbuf,
          lbe0, lbe1, lbe2, lbe3, lbd0, lbd1, lbd2, lbd3,
          mb0, mb1, mb2, mb3,
          ise0, ise1, ise2, ise3, isd0, isd1, isd2, isd3,
          gs0, gs1, gs2, gs3):
        lbe = [lbe0, lbe1, lbe2, lbe3]
        lbd = [lbd0, lbd1, lbd2, lbd3]
        mb = [mb0, mb1, mb2, mb3]
        ise = [ise0, ise1, ise2, ise3]
        isd = [isd0, isd1, isd2, isd3]
        gs = [gs0, gs1, gs2, gs3]

        wid = lax.axis_index("s") * 2 + lax.axis_index("c")
        lo = wid * _NPW
        iot = _iota16()
        lo16 = jnp.full((16,), lo, jnp.int32)
        hi16 = lo16 + _NPW
        one16 = jnp.full((16,), 1, jnp.int32)
        two16 = jnp.full((16,), 2, jnp.int32)
        neg = jnp.full((16,), _NEG, jnp.float32)
        zf = jnp.zeros((16,), jnp.float32)
        zero16 = jnp.zeros((16,), jnp.int32)
        col16 = [jnp.full((16,), c * 16, jnp.int32) + iot
                 for c in range(_D // 16)]

        def initrow(r, _):
            for c in range(_D // 16):
                agg[r, pl.ds(c * 16, 16)] = neg
            for c in range(_D // 16):
                agg2[r, pl.ds(c * 16, 16)] = neg
            return 0

        lax.fori_loop(0, _NPW, initrow, 0)

        pltpu.sync_copy(cnt_hbm.at[pl.ds(wid * 16, 16)], cbuf)
        kt = cbuf[pl.ds(0, 16)][0]
        nsub = lax.shift_right_logical(kt + (_RS - 1), _RSH)

        def fire_idx(j, b):
            @pl.when(j < nsub)
            def _():
                pltpu.async_copy(le_hbm.at[wid, j], lbe[b], ise[b])
                pltpu.async_copy(ld_hbm.at[wid, j], lbd[b], isd[b])

        def fire_gather(j, b):
            @pl.when(j < nsub)
            def _():
                pltpu.make_async_copy(le_hbm.at[wid, j], lbe[b],
                                      ise[b]).wait()
                pltpu.async_copy(m_hbm.at[lbe[b]], mb[b], gs[b])

        for b in range(_NSLOT):
            fire_idx(jnp.int32(b), b)
        for b in range(_NSLOT - 1):
            fire_gather(jnp.int32(b), b)

        nblk = lax.shift_right_logical(nsub + (_NSLOT - 1), 2)

        def blk(bi, _):
            for b in range(_NSLOT):
                j = bi * _NSLOT + b
                fire_gather(j + (_NSLOT - 1), (b + _NSLOT - 1) % _NSLOT)

                @pl.when(j < nsub)
                def _(b=b, j=j):
                    pltpu.make_async_copy(ld_hbm.at[wid, j], lbd[b],
                                          isd[b]).wait()
                    with jax.named_scope("segmax_gwait"):
                        pltpu.make_async_copy(m_hbm.at[lbe[b]], mb[b],
                                              gs[b]).wait()

                    def acc(i, carry):
                        pa, pb = carry
                        for u in range(2):
                            pau = pa if u == 0 else pa + two16
                            pbu = pb if u == 0 else pb + two16
                            dva = plsc.load_gather(lbd[b], [pau])
                            dvb = plsc.load_gather(lbd[b], [pbu])
                            oka = (dva >= lo16) & (dva < hi16)
                            okb = (dvb >= lo16) & (dvb < hi16)
                            lia = dva - lo16
                            lib = dvb - lo16
                            e0 = 4 * i + 2 * u
                            for c in range(_D // 16):
                                aia = [lia, col16[c]]
                                aib = [lib, col16[c]]
                                cura = plsc.load_gather(agg, aia, mask=oka)
                                curb = plsc.load_gather(agg2, aib, mask=okb)
                                mva = mb[b][e0, pl.ds(c * 16, 16)]
                                mvb = mb[b][e0 + 1, pl.ds(c * 16, 16)]
                                plsc.store_scatter(agg, aia,
                                                   jnp.maximum(cura, mva),
                                                   mask=oka)
                                plsc.store_scatter(agg2, aib,
                                                   jnp.maximum(curb, mvb),
                                                   mask=okb)
                        return pa + two16 + two16, pb + two16 + two16

                    with jax.named_scope("segmax_acc"):
                        lax.fori_loop(0, _RS // 4, acc, (zero16, one16))

                fire_idx(j + _NSLOT, b)
            return 0

        lax.fori_loop(0, nblk, blk, 0)

        def outrow(r, _):
            for c in range(_D // 16):
                agg[r, pl.ds(c * 16, 16)] = jnp.maximum(
                    jnp.maximum(agg[r, pl.ds(c * 16, 16)],
                                agg2[r, pl.ds(c * 16, 16)]), zf)
            return 0

        lax.fori_loop(0, _NPW, outrow, 0)
        pltpu.sync_copy(agg, h_hbm.at[pl.ds(lo, _NPW)])

    return k(M, le, ld, cnt)


def _fc0_bn_pq(x, W0, b0, g, bt, Wa, Wb, be1):
    def body(x_ref, w0_ref, b0_ref, g_ref, bt_ref, wa_ref, wb_ref, be1_ref,
             p_ref, q_ref):
        h = jnp.dot(x_ref[...], w0_ref[...],
                    preferred_element_type=jnp.float32) + b0_ref[...]
        mu = jnp.mean(h, axis=0, keepdims=True)
        var = jnp.mean((h - mu) ** 2, axis=0, keepdims=True)
        hn = (h - mu) * lax.rsqrt(var + 1e-5) * g_ref[...] + bt_ref[...]
        p_ref[...] = jnp.dot(hn, wa_ref[...],
                             preferred_element_type=jnp.float32) + be1_ref[...]
        q_ref[...] = jnp.dot(hn, wb_ref[...],
                             preferred_element_type=jnp.float32)

    return pl.pallas_call(
        body,
        out_shape=(jax.ShapeDtypeStruct((_N, _D), jnp.float32),
                   jax.ShapeDtypeStruct((_N, _D), jnp.float32)),
    )(x, W0, b0, g, bt, Wa, Wb, be1)


def _bn_pq(h, g, bt, Wa, Wb, be1):
    def body(h_ref, g_ref, bt_ref, wa_ref, wb_ref, be1_ref, p_ref, q_ref):
        h = h_ref[...]
        mu = jnp.mean(h, axis=0, keepdims=True)
        var = jnp.mean((h - mu) ** 2, axis=0, keepdims=True)
        hn = (h - mu) * lax.rsqrt(var + 1e-5) * g_ref[...] + bt_ref[...]
        p_ref[...] = jnp.dot(hn, wa_ref[...],
                             preferred_element_type=jnp.float32) + be1_ref[...]
        q_ref[...] = jnp.dot(hn, wb_ref[...],
                             preferred_element_type=jnp.float32)

    return pl.pallas_call(
        body,
        out_shape=(jax.ShapeDtypeStruct((_N, _D), jnp.float32),
                   jax.ShapeDtypeStruct((_N, _D), jnp.float32)),
    )(h, g, bt, Wa, Wb, be1)


def _edge_mlp(R, We2, be2):
    br = 512
    grid = _E // br

    def body(r_ref, w_ref, b_ref, o_ref):
        h = jnp.maximum(r_ref[...], 0.0)
        o_ref[...] = jnp.dot(h, w_ref[...],
                             preferred_element_type=jnp.float32) + b_ref[...]

    return pl.pallas_call(
        body,
        grid=(grid,),
        in_specs=[pl.BlockSpec((br, _D), lambda i: (i, 0)),
                  pl.BlockSpec((_D, _D), lambda i: (0, 0)),
                  pl.BlockSpec((1, _D), lambda i: (0, 0))],
        out_specs=pl.BlockSpec((br, _D), lambda i: (i, 0)),
        out_shape=jax.ShapeDtypeStruct((_E, _D), jnp.float32),
    )(R, We2, be2)


def _head(h1, h2, batch2d, W1, b1, W2p, b2p):
    def body(h1_ref, h2_ref, b_ref, w1_ref, b1_ref, w2_ref, b2_ref, o_ref):
        bt = b_ref[...]
        gidx = lax.broadcasted_iota(jnp.int32, (_G, _N), 0)
        oh = (bt == gidx).astype(jnp.float32)
        s1 = jnp.dot(oh, h1_ref[...], preferred_element_type=jnp.float32)
        s2 = jnp.dot(oh, h2_ref[...], preferred_element_type=jnp.float32)
        cnt = jnp.maximum(jnp.sum(oh, axis=1, keepdims=True), 1.0)
        pooled = jnp.concatenate([s1, s2], axis=1) / cnt
        z = jnp.maximum(
            jnp.dot(pooled, w1_ref[...],
                    preferred_element_type=jnp.float32) + b1_ref[...], 0.0)
        lg = jnp.dot(z, w2_ref[...],
                     preferred_element_type=jnp.float32) + b2_ref[...]
        mx = jnp.max(lg, axis=1, keepdims=True)
        ls = jnp.log(jnp.sum(jnp.exp(lg - mx), axis=1, keepdims=True))
        o_ref[...] = lg - mx - ls

    return pl.pallas_call(
        body,
        out_shape=jax.ShapeDtypeStruct((_G, 128), jnp.float32),
    )(h1, h2, batch2d, W1, b1, W2p, b2p)


def kernel(x, edge_index, batch, W0, b0, bn_g0, bn_b0, We1_0, be1_0, We2_0,
           be2_0, bn_g1, bn_b1, We1_1, be1_1, We2_1, be2_1, W1, b1, W2, b2):
    f32 = jnp.float32
    src = edge_index[0]
    dst = edge_index[1]
    r2 = lambda v: v.reshape(1, -1)

    le, ld, cnt = _build_lists(dst)

    Wa0 = We1_0[:_D] - We1_0[_D:]
    Wb0 = We1_0[_D:]
    P0, Q0 = _fc0_bn_pq(x, W0, r2(b0), r2(bn_g0), r2(bn_b0), Wa0, Wb0,
                        r2(be1_0))
    R0 = _edge_gather(P0, Q0, src, dst)
    M0 = _edge_mlp(R0, We2_0, r2(be2_0))
    h1 = _segmax(M0, le, ld, cnt)[:_N]

    Wa1 = We1_1[:_D] - We1_1[_D:]
    Wb1 = We1_1[_D:]
    P1, Q1 = _bn_pq(h1, r2(bn_g1), r2(bn_b1), Wa1, Wb1, r2(be1_1))
    R1 = _edge_gather(P1, Q1, src, dst)
    M1 = _edge_mlp(R1, We2_1, r2(be2_1))
    h2 = _segmax(M1, le, ld, cnt)[:_N]

    W2p = jnp.zeros((256, 128), f32).at[:, :_NCLS].set(W2)
    b2p = jnp.full((1, 128), -1e30, f32).at[0, :_NCLS].set(b2)
    out = _head(h1, h2, batch.reshape(1, _N).astype(jnp.int32), W1, r2(b1),
                W2p, b2p)
    return out[:, :_NCLS]
